# baseline (speedup 1.0000x reference)
import jax
import jax.numpy as jnp
from jax import lax
from jax.experimental import pallas as pl
from jax.experimental.pallas import tpu as pltpu
from jax.experimental.pallas import tpu_sc as plsc

L = 32
C = 2
B = 16384
NW = 32
BW = B // NW
NG = BW // 16
GI = 4
ROWS = BW * C * L // 128

_LN2 = 0.6931471805599453
_BITS_TO_LN = _LN2 / (1 << 23)
_LN_OFFSET = (127.0 - 0.0430) * _LN2


def _bcast(ref, j):
    return plsc.load_gather(ref, [jnp.full((16,), j, jnp.int32)])


def _ln_seed(y):
    bits = plsc.bitcast(y, jnp.int32)
    return bits.astype(jnp.float32) * _BITS_TO_LN - _LN_OFFSET


def _ln_newton(y, x):
    return x - 1.0 + y * jnp.exp(-x)


def _sc_body(e_hbm, coef_hbm, out_hbm, e_v, u_v, ra_v, out_v, coef_v):
    wid = lax.axis_index("s") * 2 + lax.axis_index("c")
    base = wid * ROWS
    pltpu.sync_copy(e_hbm.at[pl.ds(base, ROWS)], e_v)
    pltpu.sync_copy(coef_hbm, coef_v)

    iota16 = lax.iota(jnp.int32, 16)
    rowc = lax.shift_right_logical(iota16, 1)
    colc = lax.shift_left(jnp.bitwise_and(iota16, 1), 6)
    ones = jnp.ones((16,), jnp.float32)

    def u_body(g, carry):
        rowv = g * 8 + rowc
        for j in range(L):
            e0 = plsc.load_gather(e_v, [rowv, colc + j])
            e1 = plsc.load_gather(e_v, [rowv, colc + (32 + j)])
            u_v[pl.ds(g * (L * 16) + j * 16, 16)] = jnp.exp(e1 - e0)
        return carry

    lax.fori_loop(0, NG, u_body, 0)

    def up_blk(gb, carry):
        g0 = gb * GI
        offs = [(g0 + gi) * (L * 16) for gi in range(GI)]
        for gi in range(GI):
            ra_v[pl.ds(offs[gi] + (L - 1) * 16, 16)] = ones
        rs = [ones] * GI
        for j in range(L - 1, 0, -1):
            c1 = _bcast(coef_v, j)
            c2 = _bcast(coef_v, L + j)
            c3 = _bcast(coef_v, 2 * L + j)
            for gi in range(GI):
                u = u_v[pl.ds(offs[gi] + j * 16, 16)]
                t = u * rs[gi]
                r2 = c1 * (1.0 + c2 * t) / (1.0 + c3 * t)
                ra_v[pl.ds(offs[gi] + (j - 1) * 16, 16)] = r2
                rs[gi] = r2
        return carry

    lax.fori_loop(0, NG // GI, up_blk, 0)

    def dn_blk(gb, carry):
        g0 = gb * GI
        offs = [(g0 + gi) * (L * 16) for gi in range(GI)]
        rowvs = [(g0 + gi) * 8 + rowc for gi in range(GI)]
        rbs = [ones] * GI
        for j in range(L):
            d1 = _bcast(coef_v, 3 * L + j)
            d2 = _bcast(coef_v, 4 * L + j)
            d3 = _bcast(coef_v, 5 * L + j)
            for gi in range(GI):
                u = u_v[pl.ds(offs[gi] + j * 16, 16)]
                raj = ra_v[pl.ds(offs[gi] + j * 16, 16)]
                t = u * rbs[gi]
                q = t * raj
                y = 1.0 + q
                x = _ln_newton(y, _ln_seed(y))
                xq = _ln_newton(q, _ln_seed(q))
                plsc.store_scatter(out_v, [rowvs[gi], colc + j], -x)
                plsc.store_scatter(out_v, [rowvs[gi], colc + (32 + j)], xq - x)
                rbs[gi] = d1 * (1.0 + d2 * t) / (1.0 + d3 * t)
        return carry

    lax.fori_loop(0, NG // GI, dn_blk, 0)

    pltpu.sync_copy(out_v, out_hbm.at[pl.ds(base, ROWS)])


@jax.jit
def _sc_call(e2, coefs):
    mesh = plsc.VectorSubcoreMesh(core_axis_name="c", subcore_axis_name="s")
    return pl.kernel(
        _sc_body,
        mesh=mesh,
        compiler_params=pltpu.CompilerParams(needs_layout_passes=False),
        out_type=jax.ShapeDtypeStruct((B * C * L // 128, 128), jnp.float32),
        scratch_types=[
            pltpu.VMEM((ROWS, 128), jnp.float32),
            pltpu.VMEM((NG * L * 16,), jnp.float32),
            pltpu.VMEM((NG * L * 16,), jnp.float32),
            pltpu.VMEM((ROWS, 128), jnp.float32),
            pltpu.VMEM((6 * L,), jnp.float32),
        ],
    )(e2, coefs)


def kernel(emissions, transitions):
    i = jnp.arange(L - 1)
    t_up = transitions[i, i + 1]
    t_dn = transitions[i + 1, i]

    def mk(t):
        return (jnp.exp(t[:, 1, 0] - t[:, 0, 0]),
                jnp.exp(t[:, 1, 1] - t[:, 1, 0]),
                jnp.exp(t[:, 0, 1] - t[:, 0, 0]))

    c1, c2, c3 = mk(t_up)
    d1, d2, d3 = mk(t_dn)
    one = jnp.ones((1,), jnp.float32)
    coefs = jnp.concatenate(
        [one, c1, one, c2, one, c3, d1, one, d2, one, d3, one])
    e2 = jnp.reshape(emissions, (B * C * L // 128, 128))
    out2 = _sc_call(e2, coefs)
    return jnp.reshape(out2, (B, C, L))


# speedup vs baseline: 1.6996x; 1.6996x over previous
import jax
import jax.numpy as jnp
from jax import lax
from jax.experimental import pallas as pl
from jax.experimental.pallas import tpu as pltpu
from jax.experimental.pallas import tpu_sc as plsc

L = 32
C = 2
B = 16384
NW = 32
BW = B // NW
NG = BW // 16
GI = 4
TCW = BW // 128

_LN2 = 0.6931471805599453
_BITS_TO_LN = _LN2 / (1 << 23)
_LN_OFFSET = (127.0 - 0.0430) * _LN2


def _bcast(ref, j):
    return plsc.load_gather(ref, [jnp.full((16,), j, jnp.int32)])


def _ln_seed(y):
    bits = plsc.bitcast(y, jnp.int32)
    return bits.astype(jnp.float32) * _BITS_TO_LN - _LN_OFFSET


def _ln_newton(y, x):
    return x - 1.0 + y * jnp.exp(-x)


def _eslice(ref, g, c, j):
    return ref[c * 4 + (j >> 3), g >> 3, j & 7, pl.ds((g & 7) * 16, 16)]


def _sc_body(e_hbm, coef_hbm, out_hbm, e_v, ra_v, out_v, coef_v):
    wid = lax.axis_index("s") * 2 + lax.axis_index("c")
    base = wid * TCW
    pltpu.sync_copy(e_hbm.at[:, pl.ds(base, TCW)], e_v)
    pltpu.sync_copy(coef_hbm, coef_v)

    ones = jnp.ones((16,), jnp.float32)

    def up_blk(gb, carry):
        g0 = gb * GI
        offs = [(g0 + gi) * (L * 16) for gi in range(GI)]
        for gi in range(GI):
            ra_v[pl.ds(offs[gi] + (L - 1) * 16, 16)] = ones
        rs = [ones] * GI
        for j in range(L - 1, 0, -1):
            c1 = _bcast(coef_v, j)
            c2 = _bcast(coef_v, L + j)
            c3 = _bcast(coef_v, 2 * L + j)
            for gi in range(GI):
                g = g0 + gi
                u = jnp.exp(_eslice(e_v, g, 1, j) - _eslice(e_v, g, 0, j))
                t = u * rs[gi]
                r2 = c1 * (1.0 + c2 * t) / (1.0 + c3 * t)
                ra_v[pl.ds(offs[gi] + (j - 1) * 16, 16)] = r2
                rs[gi] = r2
        return carry

    lax.fori_loop(0, NG // GI, up_blk, 0)

    def dn_blk(gb, carry):
        g0 = gb * GI
        offs = [(g0 + gi) * (L * 16) for gi in range(GI)]
        rbs = [ones] * GI
        for j in range(L):
            d1 = _bcast(coef_v, 3 * L + j)
            d2 = _bcast(coef_v, 4 * L + j)
            d3 = _bcast(coef_v, 5 * L + j)
            for gi in range(GI):
                g = g0 + gi
                u = jnp.exp(_eslice(e_v, g, 1, j) - _eslice(e_v, g, 0, j))
                raj = ra_v[pl.ds(offs[gi] + j * 16, 16)]
                t = u * rbs[gi]
                q = t * raj
                y = 1.0 + q
                x = _ln_newton(y, _ln_seed(y))
                xq = _ln_newton(q, _ln_seed(q))
                out_v[0 * 4 + (j >> 3), g >> 3, j & 7,
                      pl.ds((g & 7) * 16, 16)] = -x
                out_v[1 * 4 + (j >> 3), g >> 3, j & 7,
                      pl.ds((g & 7) * 16, 16)] = xq - x
                rbs[gi] = d1 * (1.0 + d2 * t) / (1.0 + d3 * t)
        return carry

    lax.fori_loop(0, NG // GI, dn_blk, 0)

    pltpu.sync_copy(out_v, out_hbm.at[:, pl.ds(base, TCW)])


@jax.jit
def _sc_call(e_sc, coefs):
    mesh = plsc.VectorSubcoreMesh(core_axis_name="c", subcore_axis_name="s")
    return pl.kernel(
        _sc_body,
        mesh=mesh,
        compiler_params=pltpu.CompilerParams(needs_layout_passes=False),
        out_type=jax.ShapeDtypeStruct((C * L // 8, B // 128, 8, 128),
                                      jnp.float32),
        scratch_types=[
            pltpu.VMEM((C * L // 8, TCW, 8, 128), jnp.float32),
            pltpu.VMEM((NG * L * 16,), jnp.float32),
            pltpu.VMEM((C * L // 8, TCW, 8, 128), jnp.float32),
            pltpu.VMEM((6 * L,), jnp.float32),
        ],
    )(e_sc, coefs)


def kernel(emissions, transitions):
    i = jnp.arange(L - 1)
    t_up = transitions[i, i + 1]
    t_dn = transitions[i + 1, i]

    def mk(t):
        return (jnp.exp(t[:, 1, 0] - t[:, 0, 0]),
                jnp.exp(t[:, 1, 1] - t[:, 1, 0]),
                jnp.exp(t[:, 0, 1] - t[:, 0, 0]))

    c1, c2, c3 = mk(t_up)
    d1, d2, d3 = mk(t_dn)
    one = jnp.ones((1,), jnp.float32)
    coefs = jnp.concatenate(
        [one, c1, one, c2, one, c3, d1, one, d2, one, d3, one])

    e_sc = (emissions.transpose(1, 2, 0)
            .reshape(C, L // 8, 8, B // 128, 128)
            .transpose(0, 1, 3, 2, 4)
            .reshape(C * L // 8, B // 128, 8, 128))
    out_sc = _sc_call(e_sc, coefs)
    return (out_sc.reshape(C, L // 8, B // 128, 8, 128)
            .transpose(0, 1, 3, 2, 4)
            .reshape(C, L, B)
            .transpose(2, 0, 1))


# speedup vs baseline: 2.3057x; 1.3566x over previous
import jax
import jax.numpy as jnp
from jax import lax
from jax.experimental import pallas as pl
from jax.experimental.pallas import tpu as pltpu
from jax.experimental.pallas import tpu_sc as plsc

L = 32
C = 2
B = 16384
NW = 32
BW = B // NW
NG = BW // 16
GI = 8
TCW = BW // 128

_LN2 = 0.6931471805599453
_BITS_TO_LN = _LN2 / (1 << 23)
_LN_OFFSET = (127.0 - 0.0430) * _LN2


def _bcast(ref, j):
    return plsc.load_gather(ref, [jnp.full((16,), j, jnp.int32)])


def _ln_seed(y):
    bits = plsc.bitcast(y, jnp.int32)
    return bits.astype(jnp.float32) * _BITS_TO_LN - _LN_OFFSET


def _ln_newton(y, x):
    return x - 1.0 + y * jnp.exp(-x)


def _eslice(ref, g, c, jb, k):
    return ref[c * 4 + jb, g >> 3, k, pl.ds((g & 7) * 16, 16)]


def _sc_body(e_hbm, coef_hbm, out_hbm, e_v, ra_v, u_v, out_v, coef_v):
    wid = lax.axis_index("s") * 2 + lax.axis_index("c")
    base = wid * TCW
    pltpu.sync_copy(e_hbm.at[:, pl.ds(base, TCW)], e_v)
    pltpu.sync_copy(coef_hbm, coef_v)

    ones = jnp.ones((16,), jnp.float32)

    def up_step(jb, k, g0, offs, rs, last):
        j = jb * 8 + k
        cs = (_bcast(coef_v, j), _bcast(coef_v, L + j),
              _bcast(coef_v, 2 * L + j)) if not last else None
        for gi in range(GI):
            g = g0 + gi
            u = jnp.exp(_eslice(e_v, g, 1, jb, k) - _eslice(e_v, g, 0, jb, k))
            u_v[pl.ds(offs[gi] + j * 16, 16)] = u
            if not last:
                c1, c2, c3 = cs
                t = u * rs[gi]
                r2 = c1 * (1.0 + c2 * t) / (1.0 + c3 * t)
                ra_v[pl.ds(offs[gi] + (j - 1) * 16, 16)] = r2
                rs[gi] = r2
        return rs

    def up_blk(gb, carry):
        g0 = gb * GI
        offs = [(g0 + gi) * (L * 16) for gi in range(GI)]
        for gi in range(GI):
            ra_v[pl.ds(offs[gi] + (L - 1) * 16, 16)] = ones

        def up_jb(i, rs_t):
            jb = 3 - i
            rs = list(rs_t)
            for k in range(7, -1, -1):
                rs = up_step(jb, k, g0, offs, rs, last=False)
            return tuple(rs)

        rs = list(lax.fori_loop(0, 3, up_jb, (ones,) * GI))
        for k in range(7, -1, -1):
            rs = up_step(0, k, g0, offs, rs, last=(k == 0))
        return carry

    lax.fori_loop(0, NG // GI, up_blk, 0)

    def dn_blk(gb, carry):
        g0 = gb * GI
        offs = [(g0 + gi) * (L * 16) for gi in range(GI)]

        def dn_jb(jb, rbs_t):
            rbs = list(rbs_t)
            for k in range(8):
                j = jb * 8 + k
                d1 = _bcast(coef_v, 3 * L + j)
                d2 = _bcast(coef_v, 4 * L + j)
                d3 = _bcast(coef_v, 5 * L + j)
                for gi in range(GI):
                    g = g0 + gi
                    u = u_v[pl.ds(offs[gi] + j * 16, 16)]
                    raj = ra_v[pl.ds(offs[gi] + j * 16, 16)]
                    t = u * rbs[gi]
                    q = t * raj
                    y = 1.0 + q
                    x = _ln_newton(y, _ln_seed(y))
                    xq = _ln_newton(q, _ln_seed(q))
                    out_v[0 * 4 + jb, g >> 3, k,
                          pl.ds((g & 7) * 16, 16)] = -x
                    out_v[4 + jb, g >> 3, k,
                          pl.ds((g & 7) * 16, 16)] = xq - x
                    rbs[gi] = d1 * (1.0 + d2 * t) / (1.0 + d3 * t)
            return tuple(rbs)

        lax.fori_loop(0, 4, dn_jb, (ones,) * GI)
        return carry

    lax.fori_loop(0, NG // GI, dn_blk, 0)

    pltpu.sync_copy(out_v, out_hbm.at[:, pl.ds(base, TCW)])


@jax.jit
def _sc_call(e_sc, coefs):
    mesh = plsc.VectorSubcoreMesh(core_axis_name="c", subcore_axis_name="s")
    return pl.kernel(
        _sc_body,
        mesh=mesh,
        compiler_params=pltpu.CompilerParams(needs_layout_passes=False),
        out_type=jax.ShapeDtypeStruct((C * L // 8, B // 128, 8, 128),
                                      jnp.float32),
        scratch_types=[
            pltpu.VMEM((C * L // 8, TCW, 8, 128), jnp.float32),
            pltpu.VMEM((NG * L * 16,), jnp.float32),
            pltpu.VMEM((NG * L * 16,), jnp.float32),
            pltpu.VMEM((C * L // 8, TCW, 8, 128), jnp.float32),
            pltpu.VMEM((6 * L,), jnp.float32),
        ],
    )(e_sc, coefs)


def kernel(emissions, transitions):
    i = jnp.arange(L - 1)
    t_up = transitions[i, i + 1]
    t_dn = transitions[i + 1, i]

    def mk(t):
        return (jnp.exp(t[:, 1, 0] - t[:, 0, 0]),
                jnp.exp(t[:, 1, 1] - t[:, 1, 0]),
                jnp.exp(t[:, 0, 1] - t[:, 0, 0]))

    c1, c2, c3 = mk(t_up)
    d1, d2, d3 = mk(t_dn)
    one = jnp.ones((1,), jnp.float32)
    coefs = jnp.concatenate(
        [one, c1, one, c2, one, c3, d1, one, d2, one, d3, one])

    e_sc = (emissions.transpose(1, 2, 0)
            .reshape(C, L // 8, 8, B // 128, 128)
            .transpose(0, 1, 3, 2, 4)
            .reshape(C * L // 8, B // 128, 8, 128))
    out_sc = _sc_call(e_sc, coefs)
    return (out_sc.reshape(C, L // 8, B // 128, 8, 128)
            .transpose(0, 1, 3, 2, 4)
            .reshape(C, L, B)
            .transpose(2, 0, 1))
